# baseline (speedup 1.0000x reference)
import functools

import jax
import jax.numpy as jnp
from jax import lax
from jax.experimental import pallas as pl
from jax.experimental.pallas import tpu as pltpu
from jax.experimental.pallas import tpu_sc as plsc

N_NODES_C = 10000
D_C = 128
N_EDGES_C = 320000

_NC = 2
_NS = 16
_NW = _NC * _NS
_EPW = N_EDGES_C // _NW
_EBLK = 80
_NBLK = _EPW // _EBLK
_RPT = N_NODES_C // _NS

_vector_mesh = plsc.VectorSubcoreMesh(
    core_axis_name="core", subcore_axis_name="subcore")


def _deg_body(ei_hbm, out_hbm, ibuf_v, ones_v, zbuf_v, isems, ssems, deg_sh):
    c = lax.axis_index("core")
    s = lax.axis_index("subcore")
    w = c * _NS + s
    nblk = _NBLK_LO + jnp.where(w < _NBLK_XTRA, 1, 0)

    def idx_start(t):
        pltpu.make_async_copy(
            ei_hbm.at[pl.ds(0, 2), pl.ds((w + _NW * t) * _BLK, _BLK)],
            ibuf_v.at[t & 3], isems.at[t & 3]).start()

    def idx_wait(t):
        pltpu.make_async_copy(
            ei_hbm.at[pl.ds(0, 2), pl.ds(0, _BLK)],
            ibuf_v.at[t & 3], isems.at[t & 3]).wait()

    def scat_start(t):
        pltpu.make_async_copy(
            ones_v, deg_sh.at[ibuf_v.at[t & 3, 1]],
            ssems.at[t & 1]).start(add=True)

    def scat_wait(t):
        pltpu.make_async_copy(
            ones_v, deg_sh.at[ibuf_v.at[t & 3, 1]],
            ssems.at[t & 1]).wait()

    idx_start(0)
    idx_start(1)

    for k in range(_BLK // 16):
        ones_v[pl.ds(16 * k, 16)] = jnp.ones((16,), jnp.float32)

    @pl.loop(0, 40)
    def _(i):
        zbuf_v[pl.ds(i * 16, 16)] = jnp.zeros((16,), jnp.float32)

    @pl.when(s < _NS - 1)
    def _():
        pltpu.sync_copy(zbuf_v.at[pl.ds(0, 624)],
                        deg_sh.at[pl.ds(s * 624, 624)])
    @pl.when(s == _NS - 1)
    def _():
        pltpu.sync_copy(zbuf_v, deg_sh.at[pl.ds((_NS - 1) * 624, 640)])
    plsc.subcore_barrier()

    @pl.loop(0, _NBLK_LO)
    def _(t):
        @pl.when(t >= 2)
        def _():
            scat_wait(t - 2)
        @pl.when(t + 2 < nblk)
        def _():
            idx_start(t + 2)
        idx_wait(t)
        scat_start(t)

    @pl.when(w < _NBLK_XTRA)
    def _():
        t = _NBLK_LO
        scat_wait(t - 2)
        idx_wait(t)
        scat_start(t)
    scat_wait(nblk - 2)
    scat_wait(nblk - 1)

    plsc.subcore_barrier()
    @pl.when(s < _NS - 1)
    def _():
        pltpu.sync_copy(deg_sh.at[pl.ds(s * 624, 624)],
                        zbuf_v.at[pl.ds(0, 624)])
        pltpu.sync_copy(zbuf_v.at[pl.ds(0, 624)],
                        out_hbm.at[pl.ds(c * N_NODES_C + s * 624, 624)])
    @pl.when(s == _NS - 1)
    def _():
        pltpu.sync_copy(deg_sh.at[pl.ds((_NS - 1) * 624, 640)], zbuf_v)
        pltpu.sync_copy(zbuf_v,
                        out_hbm.at[pl.ds(c * N_NODES_C + (_NS - 1) * 624, 640)])


_BLK = 128
_NBLK_TOT = N_EDGES_C // _BLK
_NBLK_LO = _NBLK_TOT // _NW
_NBLK_XTRA = _NBLK_TOT - _NBLK_LO * _NW


def _segsum_body(h_hbm, ei_hbm, z_hbm, out_hbm,
                 ibuf_v, rows_v, isems, ssems, acc_sh):
    c = lax.axis_index("core")
    s = lax.axis_index("subcore")
    w = c * _NS + s
    nblk = _NBLK_LO + jnp.where(w < _NBLK_XTRA, 1, 0)

    def idx_start(t):
        pltpu.make_async_copy(
            ei_hbm.at[pl.ds(0, 2), pl.ds((w + _NW * t) * _BLK, _BLK)],
            ibuf_v.at[t & 3], isems.at[t & 3]).start()

    def idx_wait(t):
        pltpu.make_async_copy(
            ei_hbm.at[pl.ds(0, 2), pl.ds(0, _BLK)],
            ibuf_v.at[t & 3], isems.at[t & 3]).wait()

    def gather(t):
        pltpu.sync_copy(h_hbm.at[ibuf_v.at[t & 3, 0]], rows_v.at[t & 1])

    def scat_start(t):
        pltpu.make_async_copy(
            rows_v.at[t & 1], acc_sh.at[ibuf_v.at[t & 3, 1]],
            ssems.at[t & 1]).start(add=True)

    def scat_wait(t):
        pltpu.make_async_copy(
            rows_v.at[t & 1], acc_sh.at[ibuf_v.at[t & 3, 1]],
            ssems.at[t & 1]).wait()

    idx_start(0)
    idx_start(1)

    z_v = rows_v.at[0]
    pltpu.sync_copy(z_hbm, z_v)

    rbase = s * 624

    @pl.when(s < _NS - 1)
    def _():
        for k in range(4):
            pltpu.make_async_copy(
                z_v, acc_sh.at[pl.ds(rbase + k * _BLK, _BLK)],
                ssems.at[0]).start()
        pltpu.make_async_copy(
            z_v.at[pl.ds(0, 112)], acc_sh.at[pl.ds(rbase + 512, 112)],
            ssems.at[1]).start()
        for k in range(4):
            pltpu.make_async_copy(
                z_v, acc_sh.at[pl.ds(rbase, _BLK)], ssems.at[0]).wait()
        pltpu.make_async_copy(
            z_v.at[pl.ds(0, 112)], acc_sh.at[pl.ds(rbase + 512, 112)],
            ssems.at[1]).wait()
    @pl.when(s == _NS - 1)
    def _():
        for k in range(5):
            pltpu.make_async_copy(
                z_v, acc_sh.at[pl.ds(rbase + k * _BLK, _BLK)],
                ssems.at[0]).start()
        for k in range(5):
            pltpu.make_async_copy(
                z_v, acc_sh.at[pl.ds(rbase, _BLK)], ssems.at[0]).wait()
    plsc.subcore_barrier()

    @pl.loop(0, _NBLK_LO)
    def _(t):
        @pl.when(t >= 2)
        def _():
            scat_wait(t - 2)
        @pl.when(t + 2 < nblk)
        def _():
            idx_start(t + 2)
        idx_wait(t)
        gather(t)
        scat_start(t)

    @pl.when(w < _NBLK_XTRA)
    def _():
        t = _NBLK_LO
        scat_wait(t - 2)
        idx_wait(t)
        gather(t)
        scat_start(t)
    scat_wait(nblk - 2)
    scat_wait(nblk - 1)

    plsc.subcore_barrier()

    def co_in(k):
        pltpu.sync_copy(acc_sh.at[pl.ds(rbase + k * _BLK, _BLK)],
                        rows_v.at[k & 1])

    def co_out_start(k):
        pltpu.make_async_copy(rows_v.at[k & 1],
                              out_hbm.at[c, pl.ds(rbase + k * _BLK, _BLK)],
                              ssems.at[k & 1]).start()

    def co_out_wait(k):
        pltpu.make_async_copy(rows_v.at[k & 1],
                              out_hbm.at[c, pl.ds(rbase + k * _BLK, _BLK)],
                              ssems.at[k & 1]).wait()

    @pl.when(s < _NS - 1)
    def _():
        for k in range(4):
            if k >= 2:
                co_out_wait(k - 2)
            co_in(k)
            co_out_start(k)
        co_out_wait(2)
        pltpu.sync_copy(acc_sh.at[pl.ds(rbase + 512, 112)],
                        rows_v.at[0].at[pl.ds(0, 112)])
        pltpu.make_async_copy(
            rows_v.at[0].at[pl.ds(0, 112)],
            out_hbm.at[c, pl.ds(rbase + 512, 112)], ssems.at[0]).start()
        co_out_wait(3)
        pltpu.make_async_copy(
            rows_v.at[0].at[pl.ds(0, 112)],
            out_hbm.at[c, pl.ds(rbase + 512, 112)], ssems.at[0]).wait()
    @pl.when(s == _NS - 1)
    def _():
        for k in range(5):
            if k >= 2:
                co_out_wait(k - 2)
            co_in(k)
            co_out_start(k)
        co_out_wait(3)
        co_out_wait(4)


def _sc_degree(ei):
    k = pl.kernel(
        _deg_body,
        out_type=jax.ShapeDtypeStruct((_NC * N_NODES_C,), jnp.float32),
        mesh=_vector_mesh,
        scratch_types=[
            pltpu.VMEM((4, 2, _BLK), jnp.int32),
            pltpu.VMEM((_BLK,), jnp.float32),
            pltpu.VMEM((640,), jnp.float32),
            pltpu.SemaphoreType.DMA((4,)),
            pltpu.SemaphoreType.DMA((2,)),
            pltpu.VMEM_SHARED((N_NODES_C,), jnp.float32),
        ],
    )
    return k(ei)


def _sc_segsum(h, ei, z):
    k = pl.kernel(
        _segsum_body,
        out_type=jax.ShapeDtypeStruct((_NC, N_NODES_C, D_C), jnp.float32),
        mesh=_vector_mesh,
        scratch_types=[
            pltpu.VMEM((4, 2, _BLK), jnp.int32),
            pltpu.VMEM((2, _BLK, D_C), jnp.float32),
            pltpu.SemaphoreType.DMA((4,)),
            pltpu.SemaphoreType.DMA((2,)),
            pltpu.VMEM_SHARED((N_NODES_C, D_C), jnp.float32),
        ],
    )
    return k(h, ei, z)


def _dinv(degp_ref):
    deg = degp_ref[:, 0:1] + degp_ref[:, 1:2]
    return jnp.where(deg > 0.0,
                     lax.rsqrt(jnp.maximum(deg, 1e-12)), 0.0)


def _tc1_body(x_ref, w1_ref, degp_ref, hs_ref):
    h = jnp.dot(x_ref[...], w1_ref[...],
                preferred_element_type=jnp.float32,
                precision=lax.Precision.HIGHEST)
    hs_ref[...] = h * _dinv(degp_ref)


def _tc2_body(s1_ref, degp_ref, b1_ref, w2_ref, h1_ref, gs_ref):
    dinv = _dinv(degp_ref)
    a1 = (s1_ref[0] + s1_ref[1]) * dinv + b1_ref[...]
    h1 = jnp.maximum(a1, 0.0)
    h1_ref[...] = h1
    g = jnp.dot(h1, w2_ref[...],
                preferred_element_type=jnp.float32,
                precision=lax.Precision.HIGHEST)
    gs_ref[...] = g * dinv


def _tc3_body(s2_ref, degp_ref, b2_ref, h1_ref, ws_ref, bs_ref, out_ref):
    dinv = _dinv(degp_ref)
    a2 = (s2_ref[0] + s2_ref[1]) * dinv + b2_ref[...]
    h2 = jnp.maximum(a2, 0.0) + h1_ref[...]
    out_ref[...] = jnp.dot(h2, ws_ref[...],
                           preferred_element_type=jnp.float32,
                           precision=lax.Precision.HIGHEST) + bs_ref[...]


_f32 = jnp.float32


def _tc1(x, W1, degp):
    return pl.pallas_call(
        _tc1_body,
        out_shape=jax.ShapeDtypeStruct((N_NODES_C, D_C), _f32),
    )(x, W1, degp)


def _tc2(s1, degp, b1, W2):
    return pl.pallas_call(
        _tc2_body,
        out_shape=(jax.ShapeDtypeStruct((N_NODES_C, D_C), _f32),
                   jax.ShapeDtypeStruct((N_NODES_C, D_C), _f32)),
    )(s1, degp, b1, W2)


def _tc3(s2, degp, b2, h1, Ws, bs):
    return pl.pallas_call(
        _tc3_body,
        out_shape=jax.ShapeDtypeStruct((N_NODES_C, D_C), _f32),
    )(s2, degp, b2, h1, Ws, bs)


@jax.jit
def kernel(x, edge_index, W1, b1, W2, b2, Ws, bs):
    z = jnp.zeros((_BLK, D_C), _f32)
    degp = _sc_degree(edge_index).reshape(_NC, N_NODES_C).T
    hs = _tc1(x, W1, degp)
    s1 = _sc_segsum(hs, edge_index, z)
    h1, gs = _tc2(s1, degp, b1, W2)
    s2 = _sc_segsum(gs, edge_index, z)
    return _tc3(s2, degp, b2, h1, Ws, bs)


# speedup vs baseline: 1.0065x; 1.0065x over previous
import functools

import jax
import jax.numpy as jnp
from jax import lax
from jax.experimental import pallas as pl
from jax.experimental.pallas import tpu as pltpu
from jax.experimental.pallas import tpu_sc as plsc

N_NODES_C = 10000
D_C = 128
N_EDGES_C = 320000

_NC = 2
_NS = 16
_NW = _NC * _NS
_EPW = N_EDGES_C // _NW
_EBLK = 80
_NBLK = _EPW // _EBLK
_RPT = N_NODES_C // _NS

_vector_mesh = plsc.VectorSubcoreMesh(
    core_axis_name="core", subcore_axis_name="subcore")


def _deg_body(ei_hbm, out_hbm, ibuf_v, ones_v, zbuf_v, isems, ssems, deg_sh):
    c = lax.axis_index("core")
    s = lax.axis_index("subcore")
    w = c * _NS + s
    nblk = _NBLK_LO + jnp.where(w < _NBLK_XTRA, 1, 0)

    def idx_start(t):
        pltpu.make_async_copy(
            ei_hbm.at[pl.ds(0, 2), pl.ds((w + _NW * t) * _BLK, _BLK)],
            ibuf_v.at[t & 3], isems.at[t & 3]).start()

    def idx_wait(t):
        pltpu.make_async_copy(
            ei_hbm.at[pl.ds(0, 2), pl.ds(0, _BLK)],
            ibuf_v.at[t & 3], isems.at[t & 3]).wait()

    def scat_start(t):
        pltpu.make_async_copy(
            ones_v, deg_sh.at[ibuf_v.at[t & 3, 1]],
            ssems.at[t & 1]).start(add=True)

    def scat_wait(t):
        pltpu.make_async_copy(
            ones_v, deg_sh.at[ibuf_v.at[t & 3, 1]],
            ssems.at[t & 1]).wait()

    idx_start(0)
    idx_start(1)

    for k in range(_BLK // 16):
        ones_v[pl.ds(16 * k, 16)] = jnp.ones((16,), jnp.float32)

    @pl.loop(0, 40)
    def _(i):
        zbuf_v[pl.ds(i * 16, 16)] = jnp.zeros((16,), jnp.float32)

    @pl.when(s < _NS - 1)
    def _():
        pltpu.sync_copy(zbuf_v.at[pl.ds(0, 624)],
                        deg_sh.at[pl.ds(s * 624, 624)])
    @pl.when(s == _NS - 1)
    def _():
        pltpu.sync_copy(zbuf_v, deg_sh.at[pl.ds((_NS - 1) * 624, 640)])
    plsc.subcore_barrier()

    @pl.loop(0, _NBLK_LO)
    def _(t):
        @pl.when(t >= 2)
        def _():
            scat_wait(t - 2)
        @pl.when(t + 2 < nblk)
        def _():
            idx_start(t + 2)
        idx_wait(t)
        scat_start(t)

    @pl.when(w < _NBLK_XTRA)
    def _():
        t = _NBLK_LO
        scat_wait(t - 2)
        idx_wait(t)
        scat_start(t)
    scat_wait(nblk - 2)
    scat_wait(nblk - 1)

    plsc.subcore_barrier()
    @pl.when(s < _NS - 1)
    def _():
        pltpu.sync_copy(deg_sh.at[pl.ds(s * 624, 624)],
                        zbuf_v.at[pl.ds(0, 624)])
        pltpu.sync_copy(zbuf_v.at[pl.ds(0, 624)],
                        out_hbm.at[pl.ds(c * N_NODES_C + s * 624, 624)])
    @pl.when(s == _NS - 1)
    def _():
        pltpu.sync_copy(deg_sh.at[pl.ds((_NS - 1) * 624, 640)], zbuf_v)
        pltpu.sync_copy(zbuf_v,
                        out_hbm.at[pl.ds(c * N_NODES_C + (_NS - 1) * 624, 640)])


_BLK = 128
_NBLK_TOT = N_EDGES_C // _BLK
_NBLK_LO = _NBLK_TOT // _NW
_NBLK_XTRA = _NBLK_TOT - _NBLK_LO * _NW


def _segsum_body(h_hbm, ei_hbm, z_hbm, out_hbm,
                 ibuf_v, rows_v, isems, ssems, acc_sh):
    c = lax.axis_index("core")
    s = lax.axis_index("subcore")
    w = c * _NS + s
    nblk = _NBLK_LO + jnp.where(w < _NBLK_XTRA, 1, 0)

    def idx_start(t):
        pltpu.make_async_copy(
            ei_hbm.at[pl.ds(0, 2), pl.ds((w + _NW * t) * _BLK, _BLK)],
            ibuf_v.at[t & 3], isems.at[t & 3]).start()

    def idx_wait(t):
        pltpu.make_async_copy(
            ei_hbm.at[pl.ds(0, 2), pl.ds(0, _BLK)],
            ibuf_v.at[t & 3], isems.at[t & 3]).wait()

    def gather(t):
        pltpu.sync_copy(h_hbm.at[ibuf_v.at[t & 3, 0]], rows_v.at[t & 1])

    def scat_start(t):
        pltpu.make_async_copy(
            rows_v.at[t & 1], acc_sh.at[ibuf_v.at[t & 3, 1]],
            ssems.at[t & 1]).start(add=True)

    def scat_wait(t):
        pltpu.make_async_copy(
            rows_v.at[t & 1], acc_sh.at[ibuf_v.at[t & 3, 1]],
            ssems.at[t & 1]).wait()

    idx_start(0)
    idx_start(1)

    z_v = rows_v.at[0]
    pltpu.sync_copy(z_hbm, z_v)

    rbase = s * 624

    @pl.when(s < _NS - 1)
    def _():
        for k in range(4):
            pltpu.make_async_copy(
                z_v, acc_sh.at[pl.ds(rbase + k * _BLK, _BLK)],
                ssems.at[0]).start()
        pltpu.make_async_copy(
            z_v.at[pl.ds(0, 112)], acc_sh.at[pl.ds(rbase + 512, 112)],
            ssems.at[1]).start()
        for k in range(4):
            pltpu.make_async_copy(
                z_v, acc_sh.at[pl.ds(rbase, _BLK)], ssems.at[0]).wait()
        pltpu.make_async_copy(
            z_v.at[pl.ds(0, 112)], acc_sh.at[pl.ds(rbase + 512, 112)],
            ssems.at[1]).wait()
    @pl.when(s == _NS - 1)
    def _():
        for k in range(5):
            pltpu.make_async_copy(
                z_v, acc_sh.at[pl.ds(rbase + k * _BLK, _BLK)],
                ssems.at[0]).start()
        for k in range(5):
            pltpu.make_async_copy(
                z_v, acc_sh.at[pl.ds(rbase, _BLK)], ssems.at[0]).wait()
    plsc.subcore_barrier()

    @pl.loop(0, _NBLK_LO)
    def _(t):
        @pl.when(t >= 2)
        def _():
            scat_wait(t - 2)
        @pl.when(t + 2 < nblk)
        def _():
            idx_start(t + 2)
        idx_wait(t)
        gather(t)
        scat_start(t)

    @pl.when(w < _NBLK_XTRA)
    def _():
        t = _NBLK_LO
        scat_wait(t - 2)
        idx_wait(t)
        gather(t)
        scat_start(t)
    scat_wait(nblk - 2)
    scat_wait(nblk - 1)

    plsc.subcore_barrier()

    def co_in(k):
        pltpu.sync_copy(acc_sh.at[pl.ds(rbase + k * _BLK, _BLK)],
                        rows_v.at[k & 1])

    def co_out_start(k):
        pltpu.make_async_copy(rows_v.at[k & 1],
                              out_hbm.at[c, pl.ds(rbase + k * _BLK, _BLK)],
                              ssems.at[k & 1]).start()

    def co_out_wait(k):
        pltpu.make_async_copy(rows_v.at[k & 1],
                              out_hbm.at[c, pl.ds(rbase + k * _BLK, _BLK)],
                              ssems.at[k & 1]).wait()

    @pl.when(s < _NS - 1)
    def _():
        for k in range(4):
            if k >= 2:
                co_out_wait(k - 2)
            co_in(k)
            co_out_start(k)
        co_out_wait(2)
        pltpu.sync_copy(acc_sh.at[pl.ds(rbase + 512, 112)],
                        rows_v.at[0].at[pl.ds(0, 112)])
        pltpu.make_async_copy(
            rows_v.at[0].at[pl.ds(0, 112)],
            out_hbm.at[c, pl.ds(rbase + 512, 112)], ssems.at[0]).start()
        co_out_wait(3)
        pltpu.make_async_copy(
            rows_v.at[0].at[pl.ds(0, 112)],
            out_hbm.at[c, pl.ds(rbase + 512, 112)], ssems.at[0]).wait()
    @pl.when(s == _NS - 1)
    def _():
        for k in range(5):
            if k >= 2:
                co_out_wait(k - 2)
            co_in(k)
            co_out_start(k)
        co_out_wait(3)
        co_out_wait(4)


def _sc_degree(ei):
    k = pl.kernel(
        _deg_body,
        out_type=jax.ShapeDtypeStruct((_NC * N_NODES_C,), jnp.float32),
        mesh=_vector_mesh,
        scratch_types=[
            pltpu.VMEM((4, 2, _BLK), jnp.int32),
            pltpu.VMEM((_BLK,), jnp.float32),
            pltpu.VMEM((640,), jnp.float32),
            pltpu.SemaphoreType.DMA((4,)),
            pltpu.SemaphoreType.DMA((2,)),
            pltpu.VMEM_SHARED((N_NODES_C,), jnp.float32),
        ],
    )
    return k(ei)


def _sc_segsum(h, ei, z):
    k = pl.kernel(
        _segsum_body,
        out_type=jax.ShapeDtypeStruct((_NC, N_NODES_C, D_C), jnp.float32),
        mesh=_vector_mesh,
        scratch_types=[
            pltpu.VMEM((4, 2, _BLK), jnp.int32),
            pltpu.VMEM((2, _BLK, D_C), jnp.float32),
            pltpu.SemaphoreType.DMA((4,)),
            pltpu.SemaphoreType.DMA((2,)),
            pltpu.VMEM_SHARED((N_NODES_C, D_C), jnp.float32),
        ],
    )
    return k(h, ei, z)


def _dinv(degp_ref):
    deg = degp_ref[:, 0:1] + degp_ref[:, 1:2]
    return jnp.where(deg > 0.0,
                     lax.rsqrt(jnp.maximum(deg, 1e-12)), 0.0)


def _tc0_body(x_ref, w1_ref, h_ref):
    h_ref[...] = jnp.dot(x_ref[...], w1_ref[...],
                         preferred_element_type=jnp.float32,
                         precision=lax.Precision.HIGHEST)


def _tc1_body(h_ref, degp_ref, hs_ref):
    hs_ref[...] = h_ref[...] * _dinv(degp_ref)


def _tc2_body(s1_ref, degp_ref, b1_ref, w2_ref, h1_ref, gs_ref):
    dinv = _dinv(degp_ref)
    a1 = (s1_ref[0] + s1_ref[1]) * dinv + b1_ref[...]
    h1 = jnp.maximum(a1, 0.0)
    h1_ref[...] = h1
    g = jnp.dot(h1, w2_ref[...],
                preferred_element_type=jnp.float32,
                precision=lax.Precision.HIGHEST)
    gs_ref[...] = g * dinv


def _tc3_body(s2_ref, degp_ref, b2_ref, h1_ref, ws_ref, bs_ref, out_ref):
    dinv = _dinv(degp_ref)
    a2 = (s2_ref[0] + s2_ref[1]) * dinv + b2_ref[...]
    h2 = jnp.maximum(a2, 0.0) + h1_ref[...]
    out_ref[...] = jnp.dot(h2, ws_ref[...],
                           preferred_element_type=jnp.float32,
                           precision=lax.Precision.HIGHEST) + bs_ref[...]


_f32 = jnp.float32


def _tc0(x, W1):
    return pl.pallas_call(
        _tc0_body,
        out_shape=jax.ShapeDtypeStruct((N_NODES_C, D_C), _f32),
    )(x, W1)


def _tc1(h, degp):
    return pl.pallas_call(
        _tc1_body,
        out_shape=jax.ShapeDtypeStruct((N_NODES_C, D_C), _f32),
    )(h, degp)


def _tc2(s1, degp, b1, W2):
    return pl.pallas_call(
        _tc2_body,
        out_shape=(jax.ShapeDtypeStruct((N_NODES_C, D_C), _f32),
                   jax.ShapeDtypeStruct((N_NODES_C, D_C), _f32)),
    )(s1, degp, b1, W2)


def _tc3(s2, degp, b2, h1, Ws, bs):
    return pl.pallas_call(
        _tc3_body,
        out_shape=jax.ShapeDtypeStruct((N_NODES_C, D_C), _f32),
    )(s2, degp, b2, h1, Ws, bs)


@jax.jit
def kernel(x, edge_index, W1, b1, W2, b2, Ws, bs):
    z = jnp.zeros((_BLK, D_C), _f32)
    h = _tc0(x, W1)
    degp = _sc_degree(edge_index).reshape(_NC, N_NODES_C).T
    hs = _tc1(h, degp)
    s1 = _sc_segsum(hs, edge_index, z)
    h1, gs = _tc2(s1, degp, b1, W2)
    s2 = _sc_segsum(gs, edge_index, z)
    return _tc3(s2, degp, b2, h1, Ws, bs)


# speedup vs baseline: 1.0147x; 1.0081x over previous
import functools

import jax
import jax.numpy as jnp
from jax import lax
from jax.experimental import pallas as pl
from jax.experimental.pallas import tpu as pltpu
from jax.experimental.pallas import tpu_sc as plsc

N_NODES_C = 10000
D_C = 128
N_EDGES_C = 320000

_NC = 2
_NS = 16
_NW = _NC * _NS
_EPW = N_EDGES_C // _NW
_EBLK = 80
_NBLK = _EPW // _EBLK
_RPT = N_NODES_C // _NS

_vector_mesh = plsc.VectorSubcoreMesh(
    core_axis_name="core", subcore_axis_name="subcore")


def _deg_body(ei_hbm, out_hbm, ibuf_v, ones_v, zbuf_v, isems, ssems, deg_sh):
    c = lax.axis_index("core")
    s = lax.axis_index("subcore")
    w = c * _NS + s
    nblk = _NBLK_LO + jnp.where(w < _NBLK_XTRA, 1, 0)

    def idx_start(t):
        pltpu.make_async_copy(
            ei_hbm.at[pl.ds(0, 2), pl.ds((w + _NW * t) * _BLK, _BLK)],
            ibuf_v.at[t & 3], isems.at[t & 3]).start()

    def idx_wait(t):
        pltpu.make_async_copy(
            ei_hbm.at[pl.ds(0, 2), pl.ds(0, _BLK)],
            ibuf_v.at[t & 3], isems.at[t & 3]).wait()

    def scat_start(t):
        pltpu.make_async_copy(
            ones_v, deg_sh.at[ibuf_v.at[t & 3, 1]],
            ssems.at[t & 1]).start(add=True)

    def scat_wait(t):
        pltpu.make_async_copy(
            ones_v, deg_sh.at[ibuf_v.at[t & 3, 1]],
            ssems.at[t & 1]).wait()

    idx_start(0)
    idx_start(1)

    for k in range(_BLK // 16):
        ones_v[pl.ds(16 * k, 16)] = jnp.ones((16,), jnp.float32)

    @pl.loop(0, 40)
    def _(i):
        zbuf_v[pl.ds(i * 16, 16)] = jnp.zeros((16,), jnp.float32)

    @pl.when(s < _NS - 1)
    def _():
        pltpu.sync_copy(zbuf_v.at[pl.ds(0, 624)],
                        deg_sh.at[pl.ds(s * 624, 624)])
    @pl.when(s == _NS - 1)
    def _():
        pltpu.sync_copy(zbuf_v, deg_sh.at[pl.ds((_NS - 1) * 624, 640)])
    plsc.subcore_barrier()

    @pl.loop(0, _NBLK_LO)
    def _(t):
        @pl.when(t >= 2)
        def _():
            scat_wait(t - 2)
        @pl.when(t + 2 < nblk)
        def _():
            idx_start(t + 2)
        idx_wait(t)
        scat_start(t)

    @pl.when(w < _NBLK_XTRA)
    def _():
        t = _NBLK_LO
        scat_wait(t - 2)
        idx_wait(t)
        scat_start(t)
    scat_wait(nblk - 2)
    scat_wait(nblk - 1)

    plsc.subcore_barrier()
    @pl.when(s < _NS - 1)
    def _():
        pltpu.sync_copy(deg_sh.at[pl.ds(s * 624, 624)],
                        zbuf_v.at[pl.ds(0, 624)])
        pltpu.sync_copy(zbuf_v.at[pl.ds(0, 624)],
                        out_hbm.at[pl.ds(c * N_NODES_C + s * 624, 624)])
    @pl.when(s == _NS - 1)
    def _():
        pltpu.sync_copy(deg_sh.at[pl.ds((_NS - 1) * 624, 640)], zbuf_v)
        pltpu.sync_copy(zbuf_v,
                        out_hbm.at[pl.ds(c * N_NODES_C + (_NS - 1) * 624, 640)])


_BLK = 128
_NBLK_TOT = N_EDGES_C // _BLK
_NBLK_LO = _NBLK_TOT // _NW
_NBLK_XTRA = _NBLK_TOT - _NBLK_LO * _NW


def _segsum_body(h_hbm, ei_hbm, z_hbm, out_hbm,
                 ibuf_v, rows_v, isems, ssems, acc_sh):
    c = lax.axis_index("core")
    s = lax.axis_index("subcore")
    w = c * _NS + s
    nblk = _NBLK_LO + jnp.where(w < _NBLK_XTRA, 1, 0)

    def idx_start(t):
        pltpu.make_async_copy(
            ei_hbm.at[pl.ds(0, 2), pl.ds((w + _NW * t) * _BLK, _BLK)],
            ibuf_v.at[t & 3], isems.at[t & 3]).start()

    def idx_wait(t):
        pltpu.make_async_copy(
            ei_hbm.at[pl.ds(0, 2), pl.ds(0, _BLK)],
            ibuf_v.at[t & 3], isems.at[t & 3]).wait()

    def gather(t):
        pltpu.sync_copy(h_hbm.at[ibuf_v.at[t & 3, 0]], rows_v.at[t & 1])

    def scat_start(t):
        pltpu.make_async_copy(
            rows_v.at[t & 1], acc_sh.at[ibuf_v.at[t & 3, 1]],
            ssems.at[t & 1]).start(add=True)

    def scat_wait(t):
        pltpu.make_async_copy(
            rows_v.at[t & 1], acc_sh.at[ibuf_v.at[t & 3, 1]],
            ssems.at[t & 1]).wait()

    idx_start(0)
    idx_start(1)

    z_v = rows_v.at[0]
    pltpu.sync_copy(z_hbm, z_v)

    rbase = s * 624

    @pl.when(s < _NS - 1)
    def _():
        for k in range(4):
            pltpu.make_async_copy(
                z_v, acc_sh.at[pl.ds(rbase + k * _BLK, _BLK)],
                ssems.at[0]).start()
        pltpu.make_async_copy(
            z_v.at[pl.ds(0, 112)], acc_sh.at[pl.ds(rbase + 512, 112)],
            ssems.at[1]).start()
        for k in range(4):
            pltpu.make_async_copy(
                z_v, acc_sh.at[pl.ds(rbase, _BLK)], ssems.at[0]).wait()
        pltpu.make_async_copy(
            z_v.at[pl.ds(0, 112)], acc_sh.at[pl.ds(rbase + 512, 112)],
            ssems.at[1]).wait()
    @pl.when(s == _NS - 1)
    def _():
        for k in range(5):
            pltpu.make_async_copy(
                z_v, acc_sh.at[pl.ds(rbase + k * _BLK, _BLK)],
                ssems.at[0]).start()
        for k in range(5):
            pltpu.make_async_copy(
                z_v, acc_sh.at[pl.ds(rbase, _BLK)], ssems.at[0]).wait()
    plsc.subcore_barrier()

    @pl.loop(0, _NBLK_LO)
    def _(t):
        @pl.when(t >= 2)
        def _():
            scat_wait(t - 2)
        @pl.when(t + 2 < nblk)
        def _():
            idx_start(t + 2)
        idx_wait(t)
        gather(t)
        scat_start(t)

    @pl.when(w < _NBLK_XTRA)
    def _():
        t = _NBLK_LO
        scat_wait(t - 2)
        idx_wait(t)
        gather(t)
        scat_start(t)
    scat_wait(nblk - 2)
    scat_wait(nblk - 1)

    plsc.subcore_barrier()

    def co_in(k):
        pltpu.sync_copy(acc_sh.at[pl.ds(rbase + k * _BLK, _BLK)],
                        rows_v.at[k & 1])

    def co_out_start(k):
        pltpu.make_async_copy(rows_v.at[k & 1],
                              out_hbm.at[c, pl.ds(rbase + k * _BLK, _BLK)],
                              ssems.at[k & 1]).start()

    def co_out_wait(k):
        pltpu.make_async_copy(rows_v.at[k & 1],
                              out_hbm.at[c, pl.ds(rbase + k * _BLK, _BLK)],
                              ssems.at[k & 1]).wait()

    @pl.when(s < _NS - 1)
    def _():
        for k in range(4):
            if k >= 2:
                co_out_wait(k - 2)
            co_in(k)
            co_out_start(k)
        co_out_wait(2)
        pltpu.sync_copy(acc_sh.at[pl.ds(rbase + 512, 112)],
                        rows_v.at[0].at[pl.ds(0, 112)])
        pltpu.make_async_copy(
            rows_v.at[0].at[pl.ds(0, 112)],
            out_hbm.at[c, pl.ds(rbase + 512, 112)], ssems.at[0]).start()
        co_out_wait(3)
        pltpu.make_async_copy(
            rows_v.at[0].at[pl.ds(0, 112)],
            out_hbm.at[c, pl.ds(rbase + 512, 112)], ssems.at[0]).wait()
    @pl.when(s == _NS - 1)
    def _():
        for k in range(5):
            if k >= 2:
                co_out_wait(k - 2)
            co_in(k)
            co_out_start(k)
        co_out_wait(3)
        co_out_wait(4)


def _sc_degree(ei):
    k = pl.kernel(
        _deg_body,
        out_type=jax.ShapeDtypeStruct((_NC * N_NODES_C,), jnp.float32),
        mesh=_vector_mesh,
        scratch_types=[
            pltpu.VMEM((4, 2, _BLK), jnp.int32),
            pltpu.VMEM((_BLK,), jnp.float32),
            pltpu.VMEM((640,), jnp.float32),
            pltpu.SemaphoreType.DMA((4,)),
            pltpu.SemaphoreType.DMA((2,)),
            pltpu.VMEM_SHARED((N_NODES_C,), jnp.float32),
        ],
    )
    return k(ei)


def _sc_segsum(h, ei, z):
    k = pl.kernel(
        _segsum_body,
        out_type=jax.ShapeDtypeStruct((_NC, N_NODES_C, D_C), jnp.float32),
        mesh=_vector_mesh,
        scratch_types=[
            pltpu.VMEM((4, 2, _BLK), jnp.int32),
            pltpu.VMEM((2, _BLK, D_C), jnp.float32),
            pltpu.SemaphoreType.DMA((4,)),
            pltpu.SemaphoreType.DMA((2,)),
            pltpu.VMEM_SHARED((N_NODES_C, D_C), jnp.float32),
        ],
    )
    return k(h, ei, z)


def _dinv(degp_ref):
    deg = degp_ref[:, 0:1] + degp_ref[:, 1:2]
    return jnp.where(deg > 0.0,
                     lax.rsqrt(jnp.maximum(deg, 1e-12)), 0.0)


def _tc0_body(x_ref, w1_ref, h_ref):
    h_ref[...] = jnp.dot(x_ref[...], w1_ref[...],
                         preferred_element_type=jnp.float32,
                         precision=lax.Precision.HIGHEST)


def _tc1_body(h_ref, degp_ref, hs_ref):
    hs_ref[...] = h_ref[...] * _dinv(degp_ref)


def _tc2_body(s1_ref, degp_ref, b1_ref, w2_ref, h1_ref, gs_ref):
    dinv = _dinv(degp_ref)
    a1 = (s1_ref[0] + s1_ref[1]) * dinv + b1_ref[...]
    h1 = jnp.maximum(a1, 0.0)
    h1_ref[...] = h1
    g = jnp.dot(h1, w2_ref[...],
                preferred_element_type=jnp.float32,
                precision=lax.Precision.HIGHEST)
    gs_ref[...] = g * dinv


def _tc3_body(s2_ref, degp_ref, b2_ref, h1_ref, ws_ref, bs_ref, out_ref):
    dinv = _dinv(degp_ref)
    a2 = (s2_ref[0] + s2_ref[1]) * dinv + b2_ref[...]
    h2 = jnp.maximum(a2, 0.0) + h1_ref[...]
    out_ref[...] = jnp.dot(h2, ws_ref[...],
                           preferred_element_type=jnp.float32,
                           precision=lax.Precision.HIGHEST) + bs_ref[...]


_f32 = jnp.float32


_RB = 2000
_row_spec = pl.BlockSpec((_RB, D_C), lambda i: (i, 0))
_deg_spec = pl.BlockSpec((_RB, _NC), lambda i: (i, 0))
_par_spec = pl.BlockSpec((_NC, _RB, D_C), lambda i: (0, i, 0))
_w_spec = pl.BlockSpec((D_C, D_C), lambda i: (0, 0))
_b_spec = pl.BlockSpec((D_C,), lambda i: (0,))
_GRID = N_NODES_C // _RB


def _tc0(x, W1):
    return pl.pallas_call(
        _tc0_body,
        grid=(_GRID,),
        in_specs=[_row_spec, _w_spec],
        out_specs=_row_spec,
        out_shape=jax.ShapeDtypeStruct((N_NODES_C, D_C), _f32),
    )(x, W1)


def _tc1(h, degp):
    return pl.pallas_call(
        _tc1_body,
        grid=(_GRID,),
        in_specs=[_row_spec, _deg_spec],
        out_specs=_row_spec,
        out_shape=jax.ShapeDtypeStruct((N_NODES_C, D_C), _f32),
    )(h, degp)


def _tc2(s1, degp, b1, W2):
    return pl.pallas_call(
        _tc2_body,
        grid=(_GRID,),
        in_specs=[_par_spec, _deg_spec, _b_spec, _w_spec],
        out_specs=(_row_spec, _row_spec),
        out_shape=(jax.ShapeDtypeStruct((N_NODES_C, D_C), _f32),
                   jax.ShapeDtypeStruct((N_NODES_C, D_C), _f32)),
    )(s1, degp, b1, W2)


def _tc3(s2, degp, b2, h1, Ws, bs):
    return pl.pallas_call(
        _tc3_body,
        grid=(_GRID,),
        in_specs=[_par_spec, _deg_spec, _b_spec, _row_spec, _w_spec, _b_spec],
        out_specs=_row_spec,
        out_shape=jax.ShapeDtypeStruct((N_NODES_C, D_C), _f32),
    )(s2, degp, b2, h1, Ws, bs)


@jax.jit
def kernel(x, edge_index, W1, b1, W2, b2, Ws, bs):
    z = jnp.zeros((_BLK, D_C), _f32)
    h = _tc0(x, W1)
    degp = _sc_degree(edge_index).reshape(_NC, N_NODES_C).T
    hs = _tc1(h, degp)
    s1 = _sc_segsum(hs, edge_index, z)
    h1, gs = _tc2(s1, degp, b1, W2)
    s2 = _sc_segsum(gs, edge_index, z)
    return _tc3(s2, degp, b2, h1, Ws, bs)


# speedup vs baseline: 1.2431x; 1.2251x over previous
import functools

import jax
import jax.numpy as jnp
from jax import lax
from jax.experimental import pallas as pl
from jax.experimental.pallas import tpu as pltpu
from jax.experimental.pallas import tpu_sc as plsc

N_NODES_C = 10000
D_C = 128
N_EDGES_C = 320000

_NC = 2
_NS = 16
_NW = _NC * _NS
_EPW = N_EDGES_C // _NW
_EBLK = 80
_NBLK = _EPW // _EBLK
_RPT = N_NODES_C // _NS

_vector_mesh = plsc.VectorSubcoreMesh(
    core_axis_name="core", subcore_axis_name="subcore")


def _deg_body(ei_hbm, out_hbm, ibuf_v, ones_v, zbuf_v, isems, ssems, deg_sh):
    c = lax.axis_index("core")
    s = lax.axis_index("subcore")
    w = c * _NS + s
    nblk = _NBLK_LO + jnp.where(w < _NBLK_XTRA, 1, 0)

    def idx_start(t):
        pltpu.make_async_copy(
            ei_hbm.at[pl.ds(0, 2), pl.ds((w + _NW * t) * _BLK, _BLK)],
            ibuf_v.at[t & 3], isems.at[t & 3]).start()

    def idx_wait(t):
        pltpu.make_async_copy(
            ei_hbm.at[pl.ds(0, 2), pl.ds(0, _BLK)],
            ibuf_v.at[t & 3], isems.at[t & 3]).wait()

    def scat_start(t):
        pltpu.make_async_copy(
            ones_v, deg_sh.at[ibuf_v.at[t & 3, 1]],
            ssems.at[t & 1]).start(add=True)

    def scat_wait(t):
        pltpu.make_async_copy(
            ones_v, deg_sh.at[ibuf_v.at[t & 3, 1]],
            ssems.at[t & 1]).wait()

    idx_start(0)
    idx_start(1)

    for k in range(_BLK // 16):
        ones_v[pl.ds(16 * k, 16)] = jnp.ones((16,), jnp.float32)

    @pl.loop(0, 40)
    def _(i):
        zbuf_v[pl.ds(i * 16, 16)] = jnp.zeros((16,), jnp.float32)

    @pl.when(s < _NS - 1)
    def _():
        pltpu.sync_copy(zbuf_v.at[pl.ds(0, 624)],
                        deg_sh.at[pl.ds(s * 624, 624)])
    @pl.when(s == _NS - 1)
    def _():
        pltpu.sync_copy(zbuf_v, deg_sh.at[pl.ds((_NS - 1) * 624, 640)])
    plsc.subcore_barrier()

    @pl.loop(0, _NBLK_LO)
    def _(t):
        @pl.when(t >= 2)
        def _():
            scat_wait(t - 2)
        @pl.when(t + 2 < nblk)
        def _():
            idx_start(t + 2)
        idx_wait(t)
        scat_start(t)

    @pl.when(w < _NBLK_XTRA)
    def _():
        t = _NBLK_LO
        scat_wait(t - 2)
        idx_wait(t)
        scat_start(t)
    scat_wait(nblk - 2)
    scat_wait(nblk - 1)

    plsc.subcore_barrier()
    @pl.when(s < _NS - 1)
    def _():
        pltpu.sync_copy(deg_sh.at[pl.ds(s * 624, 624)],
                        zbuf_v.at[pl.ds(0, 624)])
        pltpu.sync_copy(zbuf_v.at[pl.ds(0, 624)],
                        out_hbm.at[pl.ds(c * N_NODES_C + s * 624, 624)])
    @pl.when(s == _NS - 1)
    def _():
        pltpu.sync_copy(deg_sh.at[pl.ds((_NS - 1) * 624, 640)], zbuf_v)
        pltpu.sync_copy(zbuf_v,
                        out_hbm.at[pl.ds(c * N_NODES_C + (_NS - 1) * 624, 640)])


_BLK = 128
_NBLK_TOT = N_EDGES_C // _BLK
_NBLK_LO = _NBLK_TOT // _NW
_NBLK_XTRA = _NBLK_TOT - _NBLK_LO * _NW


def _segsum_body(h_hbm, ei_hbm, z_hbm, out_hbm,
                 ibuf_v, rows_v, isems, gsems, ssems, acc_sh):
    c = lax.axis_index("core")
    s = lax.axis_index("subcore")
    w = c * _NS + s
    nblk = _NBLK_LO + jnp.where(w < _NBLK_XTRA, 1, 0)

    def idx_start(t):
        pltpu.make_async_copy(
            ei_hbm.at[pl.ds(0, 2), pl.ds((w + _NW * t) * _BLK, _BLK)],
            ibuf_v.at[t & 3], isems.at[t & 3]).start()

    def idx_wait(t):
        pltpu.make_async_copy(
            ei_hbm.at[pl.ds(0, 2), pl.ds(0, _BLK)],
            ibuf_v.at[t & 3], isems.at[t & 3]).wait()

    def rslot(t):
        return lax.rem(t, 3)

    def gat_start(t):
        pltpu.make_async_copy(
            h_hbm.at[ibuf_v.at[t & 3, 0]], rows_v.at[rslot(t)],
            gsems.at[rslot(t)]).start()

    def gat_wait(t):
        pltpu.make_async_copy(
            h_hbm.at[ibuf_v.at[t & 3, 0]], rows_v.at[rslot(t)],
            gsems.at[rslot(t)]).wait()

    def scat_start(t):
        pltpu.make_async_copy(
            rows_v.at[rslot(t)], acc_sh.at[ibuf_v.at[t & 3, 1]],
            ssems.at[rslot(t)]).start(add=True)

    def scat_wait(t):
        pltpu.make_async_copy(
            rows_v.at[rslot(t)], acc_sh.at[ibuf_v.at[t & 3, 1]],
            ssems.at[rslot(t)]).wait()

    idx_start(0)
    idx_start(1)

    z_v = rows_v.at[0]
    pltpu.sync_copy(z_hbm, z_v)

    rbase = s * 624

    @pl.when(s < _NS - 1)
    def _():
        for k in range(4):
            pltpu.make_async_copy(
                z_v, acc_sh.at[pl.ds(rbase + k * _BLK, _BLK)],
                ssems.at[0]).start()
        pltpu.make_async_copy(
            z_v.at[pl.ds(0, 112)], acc_sh.at[pl.ds(rbase + 512, 112)],
            ssems.at[1]).start()
        for k in range(4):
            pltpu.make_async_copy(
                z_v, acc_sh.at[pl.ds(rbase, _BLK)], ssems.at[0]).wait()
        pltpu.make_async_copy(
            z_v.at[pl.ds(0, 112)], acc_sh.at[pl.ds(rbase + 512, 112)],
            ssems.at[1]).wait()
    @pl.when(s == _NS - 1)
    def _():
        for k in range(5):
            pltpu.make_async_copy(
                z_v, acc_sh.at[pl.ds(rbase + k * _BLK, _BLK)],
                ssems.at[0]).start()
        for k in range(5):
            pltpu.make_async_copy(
                z_v, acc_sh.at[pl.ds(rbase, _BLK)], ssems.at[0]).wait()
    plsc.subcore_barrier()

    idx_wait(0)
    gat_start(0)

    @pl.loop(0, _NBLK_LO)
    def _(t):
        @pl.when(t >= 2)
        def _():
            scat_wait(t - 2)
        @pl.when(t + 2 < nblk)
        def _():
            idx_start(t + 2)
        @pl.when(t + 1 < nblk)
        def _():
            idx_wait(t + 1)
            gat_start(t + 1)
        gat_wait(t)
        scat_start(t)

    @pl.when(w < _NBLK_XTRA)
    def _():
        t = _NBLK_LO
        scat_wait(t - 2)
        gat_wait(t)
        scat_start(t)
    scat_wait(nblk - 2)
    scat_wait(nblk - 1)

    plsc.subcore_barrier()

    def co_in(k):
        pltpu.sync_copy(acc_sh.at[pl.ds(rbase + k * _BLK, _BLK)],
                        rows_v.at[k & 1])

    def co_out_start(k):
        pltpu.make_async_copy(rows_v.at[k & 1],
                              out_hbm.at[c, pl.ds(rbase + k * _BLK, _BLK)],
                              ssems.at[k & 1]).start()

    def co_out_wait(k):
        pltpu.make_async_copy(rows_v.at[k & 1],
                              out_hbm.at[c, pl.ds(rbase + k * _BLK, _BLK)],
                              ssems.at[k & 1]).wait()

    @pl.when(s < _NS - 1)
    def _():
        for k in range(4):
            if k >= 2:
                co_out_wait(k - 2)
            co_in(k)
            co_out_start(k)
        co_out_wait(2)
        pltpu.sync_copy(acc_sh.at[pl.ds(rbase + 512, 112)],
                        rows_v.at[0].at[pl.ds(0, 112)])
        pltpu.make_async_copy(
            rows_v.at[0].at[pl.ds(0, 112)],
            out_hbm.at[c, pl.ds(rbase + 512, 112)], ssems.at[0]).start()
        co_out_wait(3)
        pltpu.make_async_copy(
            rows_v.at[0].at[pl.ds(0, 112)],
            out_hbm.at[c, pl.ds(rbase + 512, 112)], ssems.at[0]).wait()
    @pl.when(s == _NS - 1)
    def _():
        for k in range(5):
            if k >= 2:
                co_out_wait(k - 2)
            co_in(k)
            co_out_start(k)
        co_out_wait(3)
        co_out_wait(4)


def _sc_degree(ei):
    k = pl.kernel(
        _deg_body,
        out_type=jax.ShapeDtypeStruct((_NC * N_NODES_C,), jnp.float32),
        mesh=_vector_mesh,
        scratch_types=[
            pltpu.VMEM((4, 2, _BLK), jnp.int32),
            pltpu.VMEM((_BLK,), jnp.float32),
            pltpu.VMEM((640,), jnp.float32),
            pltpu.SemaphoreType.DMA((4,)),
            pltpu.SemaphoreType.DMA((2,)),
            pltpu.VMEM_SHARED((N_NODES_C,), jnp.float32),
        ],
    )
    return k(ei)


def _sc_segsum(h, ei, z):
    k = pl.kernel(
        _segsum_body,
        out_type=jax.ShapeDtypeStruct((_NC, N_NODES_C, D_C), jnp.float32),
        mesh=_vector_mesh,
        scratch_types=[
            pltpu.VMEM((4, 2, _BLK), jnp.int32),
            pltpu.VMEM((3, _BLK, D_C), jnp.float32),
            pltpu.SemaphoreType.DMA((4,)),
            pltpu.SemaphoreType.DMA((3,)),
            pltpu.SemaphoreType.DMA((3,)),
            pltpu.VMEM_SHARED((N_NODES_C, D_C), jnp.float32),
        ],
    )
    return k(h, ei, z)


def _dinv(degp_ref):
    deg = degp_ref[:, 0:1] + degp_ref[:, 1:2]
    return jnp.where(deg > 0.0,
                     lax.rsqrt(jnp.maximum(deg, 1e-12)), 0.0)


def _tc0_body(x_ref, w1_ref, h_ref):
    h_ref[...] = jnp.dot(x_ref[...], w1_ref[...],
                         preferred_element_type=jnp.float32,
                         precision=lax.Precision.HIGHEST)


def _tc1_body(h_ref, degp_ref, hs_ref):
    hs_ref[...] = h_ref[...] * _dinv(degp_ref)


def _tc2_body(s1_ref, degp_ref, b1_ref, w2_ref, h1_ref, gs_ref):
    dinv = _dinv(degp_ref)
    a1 = (s1_ref[0] + s1_ref[1]) * dinv + b1_ref[...]
    h1 = jnp.maximum(a1, 0.0)
    h1_ref[...] = h1
    g = jnp.dot(h1, w2_ref[...],
                preferred_element_type=jnp.float32,
                precision=lax.Precision.HIGHEST)
    gs_ref[...] = g * dinv


def _tc3_body(s2_ref, degp_ref, b2_ref, h1_ref, ws_ref, bs_ref, out_ref):
    dinv = _dinv(degp_ref)
    a2 = (s2_ref[0] + s2_ref[1]) * dinv + b2_ref[...]
    h2 = jnp.maximum(a2, 0.0) + h1_ref[...]
    out_ref[...] = jnp.dot(h2, ws_ref[...],
                           preferred_element_type=jnp.float32,
                           precision=lax.Precision.HIGHEST) + bs_ref[...]


_f32 = jnp.float32


_RB = 2000
_row_spec = pl.BlockSpec((_RB, D_C), lambda i: (i, 0))
_deg_spec = pl.BlockSpec((_RB, _NC), lambda i: (i, 0))
_par_spec = pl.BlockSpec((_NC, _RB, D_C), lambda i: (0, i, 0))
_w_spec = pl.BlockSpec((D_C, D_C), lambda i: (0, 0))
_b_spec = pl.BlockSpec((D_C,), lambda i: (0,))
_GRID = N_NODES_C // _RB


def _tc0(x, W1):
    return pl.pallas_call(
        _tc0_body,
        grid=(_GRID,),
        in_specs=[_row_spec, _w_spec],
        out_specs=_row_spec,
        out_shape=jax.ShapeDtypeStruct((N_NODES_C, D_C), _f32),
    )(x, W1)


def _tc1(h, degp):
    return pl.pallas_call(
        _tc1_body,
        grid=(_GRID,),
        in_specs=[_row_spec, _deg_spec],
        out_specs=_row_spec,
        out_shape=jax.ShapeDtypeStruct((N_NODES_C, D_C), _f32),
    )(h, degp)


def _tc2(s1, degp, b1, W2):
    return pl.pallas_call(
        _tc2_body,
        grid=(_GRID,),
        in_specs=[_par_spec, _deg_spec, _b_spec, _w_spec],
        out_specs=(_row_spec, _row_spec),
        out_shape=(jax.ShapeDtypeStruct((N_NODES_C, D_C), _f32),
                   jax.ShapeDtypeStruct((N_NODES_C, D_C), _f32)),
    )(s1, degp, b1, W2)


def _tc3(s2, degp, b2, h1, Ws, bs):
    return pl.pallas_call(
        _tc3_body,
        grid=(_GRID,),
        in_specs=[_par_spec, _deg_spec, _b_spec, _row_spec, _w_spec, _b_spec],
        out_specs=_row_spec,
        out_shape=jax.ShapeDtypeStruct((N_NODES_C, D_C), _f32),
    )(s2, degp, b2, h1, Ws, bs)


@jax.jit
def kernel(x, edge_index, W1, b1, W2, b2, Ws, bs):
    z = jnp.zeros((_BLK, D_C), _f32)
    h = _tc0(x, W1)
    degp = _sc_degree(edge_index).reshape(_NC, N_NODES_C).T
    hs = _tc1(h, degp)
    s1 = _sc_segsum(hs, edge_index, z)
    h1, gs = _tc2(s1, degp, b1, W2)
    s2 = _sc_segsum(gs, edge_index, z)
    return _tc3(s2, degp, b2, h1, Ws, bs)
